# baseline (speedup 1.0000x reference)
import jax
import jax.numpy as jnp
from jax.experimental import pallas as pl

N = 1024
D_IN = 512
D_OUT = 64


def _gcn_body(x_ref, a_ref, w_ref, b_ref, o_ref):
    support = jnp.dot(x_ref[:], w_ref[:], preferred_element_type=jnp.float32)
    o_ref[:] = jnp.dot(a_ref[:], support, preferred_element_type=jnp.float32) + b_ref[:]


def kernel(input, adj, weight, bias):
    return pl.pallas_call(
        _gcn_body,
        out_shape=jax.ShapeDtypeStruct((N, D_OUT), jnp.float32),
    )(input, adj, weight, bias.reshape(1, D_OUT))
